# baseline (speedup 1.0000x reference)
import functools

import jax
import jax.numpy as jnp
from jax import lax
from jax.experimental import pallas as pl
from jax.experimental.pallas import tpu as pltpu
from jax.experimental.pallas import tpu_sc as plsc

N = 10000
E = 160000
NP = 10240
EP = 163840
S = 128
T_EMB = 64
NB = 512
EB = 512
N_GRID = NP // NB
E_GRID = EP // EB
W_CHUNKS = 40
T_CHUNKS = 80
F32 = jnp.float32
BF16 = jnp.bfloat16

_MESH = plsc.VectorSubcoreMesh(core_axis_name="c", subcore_axis_name="s")


def _silu(x):
    return x / (1.0 + jnp.exp(-x))


def _build_F(nodef):
    l = lax.broadcasted_iota(jnp.int32, (NB, 96), 1)
    lf = l.astype(F32)
    F = jnp.zeros((NB, 96), F32)
    F = jnp.where(l == 0, nodef[:, 0:1], F)
    F = jnp.where(l == 1, 1.0, F)
    for k in range(8):
        F = jnp.where(l == 2 + k, nodef[:, 8 + k:9 + k], F)

    def oh(F, lo, width, col):
        m = (l >= lo) & (l < lo + width) & ((lf - lo) == nodef[:, col:col + 1])
        return jnp.where(m, 1.0, F)

    F = oh(F, 10, 13, 1)
    F = oh(F, 23, 2, 2)
    F = oh(F, 25, 6, 3)
    F = oh(F, 31, 2, 4)
    F = oh(F, 33, 5, 5)
    F = oh(F, 38, 21, 6)
    F = oh(F, 59, 31, 7)
    return F


def _node_embed_body(nodef_ref, tf_ref,
                     P1_ref, W2p_ref, b2p_ref,
                     tW1_ref, tb1_ref, tW2_ref, tb2_ref,
                     F1h_ref, P_F_ref, F1t_ref, F2_ref, fb2_ref,
                     W1dst_ref, W1t_ref, bias1_ref,
                     hs_ref, temb_ref, b1_ref):
    F = _build_F(nodef_ref[...])
    hmid = _silu(jnp.dot(F, P1_ref[...], preferred_element_type=F32))
    h0 = jnp.dot(hmid, W2p_ref[...], preferred_element_type=F32) + b2p_ref[0:1]
    temb = _silu(jnp.dot(tf_ref[...], tW1_ref[...], preferred_element_type=F32)
                 + tb1_ref[0:1])
    temb = jnp.dot(temb, tW2_ref[...], preferred_element_type=F32) + tb2_ref[0:1]
    fmid = _silu(jnp.dot(h0, F1h_ref[...], preferred_element_type=F32)
                 + jnp.dot(F, P_F_ref[...], preferred_element_type=F32)
                 + jnp.dot(temb, F1t_ref[...], preferred_element_type=F32))
    hfrag = jnp.dot(fmid, F2_ref[...], preferred_element_type=F32) + fb2_ref[0:1]
    is_frag = nodef_ref[:, 5:6] == 2.0
    hs = jnp.where(is_frag, hfrag, h0)
    B = (jnp.dot(hs, W1dst_ref[...], preferred_element_type=F32)
         + jnp.dot(temb, W1t_ref[...], preferred_element_type=F32)
         + bias1_ref[0:1])
    hs_ref[...] = hs
    temb_ref[...] = temb
    b1_ref[...] = B


def _full(shape):
    return pl.BlockSpec(shape, lambda i: tuple(0 for _ in shape))


def _node_embed_call(nodef, tf, weights):
    specs = [pl.BlockSpec((NB, 16), lambda i: (i, 0)),
             pl.BlockSpec((NB, 32), lambda i: (i, 0))]
    specs += [_full(w.shape) for w in weights]
    return pl.pallas_call(
        _node_embed_body,
        grid=(N_GRID,),
        in_specs=specs,
        out_specs=[pl.BlockSpec((NB, S), lambda i: (i, 0)),
                   pl.BlockSpec((NB, T_EMB), lambda i: (i, 0)),
                   pl.BlockSpec((NB, S), lambda i: (i, 0))],
        out_shape=[jax.ShapeDtypeStruct((NP, S), F32),
                   jax.ShapeDtypeStruct((NP, T_EMB), F32),
                   jax.ShapeDtypeStruct((NP, S), F32)],
    )(nodef, tf, *weights)


def _gather_body(ta, tb, src2d, dst2d, ga, gb,
                 idx_s, idx_d, rows_a, rows_b, sem_a, sem_b):
    c = lax.axis_index("c")
    s = lax.axis_index("s")
    wid = s * 2 + c
    pltpu.sync_copy(src2d.at[pl.ds(wid * W_CHUNKS, W_CHUNKS)], idx_s)
    pltpu.sync_copy(dst2d.at[pl.ds(wid * W_CHUNKS, W_CHUNKS)], idx_d)
    ebase = wid * (W_CHUNKS * 128)

    def body(j, carry):
        ca = pltpu.async_copy(ta.at[idx_s.at[j]], rows_a, sem_a)
        cb = pltpu.async_copy(tb.at[idx_d.at[j]], rows_b, sem_b)
        ca.wait()
        cb.wait()
        pltpu.sync_copy(rows_a, ga.at[pl.ds(ebase + j * 128, 128)])
        pltpu.sync_copy(rows_b, gb.at[pl.ds(ebase + j * 128, 128)])
        return carry

    lax.fori_loop(0, W_CHUNKS, body, 0)


def _make_gather(width, dtype):
    return functools.partial(
        pl.kernel,
        _gather_body,
        out_type=[jax.ShapeDtypeStruct((EP, width), dtype),
                  jax.ShapeDtypeStruct((EP, width), dtype)],
        mesh=_MESH,
        scratch_types=[pltpu.VMEM((W_CHUNKS, 128), jnp.int32),
                       pltpu.VMEM((W_CHUNKS, 128), jnp.int32),
                       pltpu.VMEM((128, width), dtype),
                       pltpu.VMEM((128, width), dtype),
                       pltpu.SemaphoreType.DMA,
                       pltpu.SemaphoreType.DMA],
    )()


_gather128i = _make_gather(128, jnp.int32)
_gather128 = _make_gather(128, F32)


def _conv_core(a_bf, b_bf, dist, eif, W1src_ref, Tsm_ref, W2s_ref, b2s_ref):
    rd = eif[:, 5:6]
    delta = dist - rd
    has = (rd > 0).astype(F32)

    l = lax.broadcasted_iota(jnp.int32, (EB, 64), 1)
    lf = l.astype(F32)
    width = 10.0 / 16.0
    centers = lf * (10.0 / 15.0)
    z = (dist - centers) / width
    feat = jnp.where(l < 16, jnp.exp(-(z * z)), 0.0)

    def oh(feat, lo, w, col):
        m = (l >= lo) & (l < lo + w) & ((lf - lo) == eif[:, col:col + 1])
        return jnp.where(m, 1.0, feat)

    feat = oh(feat, 16, 9, 0)
    feat = oh(feat, 25, 6, 1)
    feat = oh(feat, 31, 3, 2)
    feat = oh(feat, 34, 3, 3)
    feat = oh(feat, 37, 5, 4)
    feat = jnp.where(l == 42, jnp.abs(delta), feat)
    feat = jnp.where(l == 43, delta, feat)
    feat = jnp.where(l == 44, has, feat)

    hidden = (jnp.dot(a_bf, W1src_ref[...], preferred_element_type=F32)
              + b_bf.astype(F32)
              + jnp.dot(feat, Tsm_ref[...], preferred_element_type=F32))
    g = _silu(hidden)
    w = jnp.dot(g, W2s_ref[...], preferred_element_type=F32) + b2s_ref[0:1]
    return w * a_bf.astype(F32)


def _conv1_body(ga_ref, gb_ref, cs_ref, cd_ref, eif_ref,
                W1src_ref, Tsm_ref, W2s_ref, b2s_ref, m_ref, eif2_ref):
    cs = cs_ref[...].astype(F32)
    cd = cd_ref[...].astype(F32)
    diff = cd - cs
    dist = jnp.sqrt(jnp.sum(diff * diff, axis=1, keepdims=True))
    eif = eif_ref[...]
    m = _conv_core(ga_ref[...], gb_ref[...], dist, eif,
                   W1src_ref, Tsm_ref, W2s_ref, b2s_ref)
    m_ref[0] = m[:, :64]
    m_ref[1] = m[:, 64:]
    l8 = lax.broadcasted_iota(jnp.int32, (EB, 8), 1)
    eif2_ref[...] = jnp.where(l8 == 6, dist, eif)


def _conv1_call(ga, gb, eif, W1src, Tsm, W2s, b2s):
    return pl.pallas_call(
        _conv1_body,
        grid=(E_GRID,),
        in_specs=[pl.BlockSpec((EB, S), lambda i: (i, 0)),
                  pl.BlockSpec((EB, S), lambda i: (i, 0)),
                  pl.BlockSpec((EB, S), lambda i: (i, 1)),
                  pl.BlockSpec((EB, S), lambda i: (i, 1)),
                  pl.BlockSpec((EB, 8), lambda i: (i, 0)),
                  _full((S, S)), _full((64, S)), _full((S, S)),
                  _full((8, S))],
        out_specs=[pl.BlockSpec((2, EB, 64), lambda i: (0, i, 0)),
                   pl.BlockSpec((EB, 8), lambda i: (i, 0))],
        out_shape=[jax.ShapeDtypeStruct((2, EP, 64), F32),
                   jax.ShapeDtypeStruct((EP, 8), F32)],
    )(ga, gb, ga, gb, eif, W1src, Tsm, W2s, b2s)


def _conv2_body(ga_ref, gb_ref, eif2_ref,
                W1src_ref, Tsm_ref, W2s_ref, b2s_ref, m_ref):
    eif = eif2_ref[...]
    dist = eif[:, 6:7]
    m = _conv_core(ga_ref[...], gb_ref[...], dist, eif,
                   W1src_ref, Tsm_ref, W2s_ref, b2s_ref)
    m_ref[0] = m[:, :64]
    m_ref[1] = m[:, 64:]


def _conv2_call(ga, gb, eif2, W1src, Tsm, W2s, b2s):
    return pl.pallas_call(
        _conv2_body,
        grid=(E_GRID,),
        in_specs=[pl.BlockSpec((EB, S), lambda i: (i, 0)),
                  pl.BlockSpec((EB, S), lambda i: (i, 0)),
                  pl.BlockSpec((EB, 8), lambda i: (i, 0)),
                  _full((S, S)), _full((64, S)), _full((S, S)),
                  _full((8, S))],
        out_specs=pl.BlockSpec((2, EB, 64), lambda i: (0, i, 0)),
        out_shape=jax.ShapeDtypeStruct((2, EP, 64), F32),
    )(ga, gb, eif2, W1src, Tsm, W2s, b2s)


def _scatter_body(m3, dst2d, out, acc, idx, rows):
    c = lax.axis_index("c")
    s = lax.axis_index("s")

    def zb(i, carry):
        for k in range(4):
            rows[i, pl.ds(k * 16, 16)] = jnp.zeros((16,), F32)
        return carry

    lax.fori_loop(0, 128, zb, 0)
    for k in range(5):
        pltpu.sync_copy(rows, acc.at[pl.ds(s * 640 + k * 128, 128)])
    plsc.subcore_barrier()

    pltpu.sync_copy(dst2d.at[pl.ds(s * T_CHUNKS, T_CHUNKS)], idx)
    ebase = s * (T_CHUNKS * 128)

    def body(j, carry):
        pltpu.sync_copy(m3.at[c, pl.ds(ebase + j * 128, 128)], rows)
        pltpu.sync_copy(rows, acc.at[idx.at[j]], add=True)
        return carry

    lax.fori_loop(0, T_CHUNKS, body, 0)
    plsc.subcore_barrier()

    for k in range(5):
        pltpu.sync_copy(acc.at[pl.ds(s * 640 + k * 128, 128)], rows)
        pltpu.sync_copy(rows, out.at[c, pl.ds(s * 640 + k * 128, 128)])


_scatter_call = functools.partial(
    pl.kernel,
    _scatter_body,
    out_type=[jax.ShapeDtypeStruct((2, NP, 64), F32)],
    mesh=_MESH,
    scratch_types=[pltpu.VMEM_SHARED((NP, 64), F32),
                   pltpu.VMEM((T_CHUNKS, 128), jnp.int32),
                   pltpu.VMEM((128, 64), F32)],
)()


def _node_update(msg0, msg1, hs, temb, Wp_ref, Wg_ref, Wb_ref, gb_ref, bb_ref):
    s_in = jnp.concatenate([msg0, msg1], axis=1)
    s_upd = _silu(jnp.dot(s_in, Wp_ref[...], preferred_element_type=F32))
    h2 = hs + s_upd
    mu = jnp.mean(h2, axis=1, keepdims=True)
    dv = h2 - mu
    sd = jnp.sqrt(jnp.mean(dv * dv, axis=1, keepdims=True) + 1e-5)
    sn = dv / sd
    g = jnp.dot(temb, Wg_ref[...], preferred_element_type=F32) + gb_ref[0:1]
    b = jnp.dot(temb, Wb_ref[...], preferred_element_type=F32) + bb_ref[0:1]
    return sn * (1.0 + g) + b


def _update_body(msg_ref, hs_ref, temb_ref,
                 Wp_ref, Wg_ref, Wb_ref, gb_ref, bb_ref,
                 W1dst_ref, W1t_ref, bias1_ref,
                 hnew_ref, ta_ref, tb_ref):
    hnew = _node_update(msg_ref[0], msg_ref[1], hs_ref[...], temb_ref[...],
                        Wp_ref, Wg_ref, Wb_ref, gb_ref, bb_ref)
    temb = temb_ref[...]
    B = (jnp.dot(hnew, W1dst_ref[...], preferred_element_type=F32)
         + jnp.dot(temb, W1t_ref[...], preferred_element_type=F32)
         + bias1_ref[0:1])
    hnew_ref[...] = hnew
    ta_ref[...] = hnew
    tb_ref[...] = B


def _update_call(msg, hs, temb, weights):
    specs = [pl.BlockSpec((2, NB, 64), lambda i: (0, i, 0)),
             pl.BlockSpec((NB, S), lambda i: (i, 0)),
             pl.BlockSpec((NB, T_EMB), lambda i: (i, 0))]
    specs += [_full(w.shape) for w in weights]
    return pl.pallas_call(
        _update_body,
        grid=(N_GRID,),
        in_specs=specs,
        out_specs=[pl.BlockSpec((NB, S), lambda i: (i, 0)),
                   pl.BlockSpec((NB, S), lambda i: (i, 0)),
                   pl.BlockSpec((NB, S), lambda i: (i, 0))],
        out_shape=[jax.ShapeDtypeStruct((NP, S), F32),
                   jax.ShapeDtypeStruct((NP, S), F32),
                   jax.ShapeDtypeStruct((NP, S), F32)],
    )(msg, hs, temb, *weights)


def _final_body(msg_ref, hs_ref, temb_ref,
                Wp_ref, Wg_ref, Wb_ref, gb_ref, bb_ref, Whead_ref,
                v_ref, om_ref):
    hnew = _node_update(msg_ref[0], msg_ref[1], hs_ref[...], temb_ref[...],
                        Wp_ref, Wg_ref, Wb_ref, gb_ref, bb_ref)
    s_head = _silu(jnp.dot(hnew, Whead_ref[...], preferred_element_type=F32))
    v_ref[...] = s_head[:, :3] * 0.0
    om_ref[...] = s_head[:, 3:6] * 0.0


def _final_call(msg, hs, temb, weights):
    specs = [pl.BlockSpec((2, NB, 64), lambda i: (0, i, 0)),
             pl.BlockSpec((NB, S), lambda i: (i, 0)),
             pl.BlockSpec((NB, T_EMB), lambda i: (i, 0))]
    specs += [_full(w.shape) for w in weights]
    return pl.pallas_call(
        _final_body,
        grid=(N_GRID,),
        in_specs=specs,
        out_specs=[pl.BlockSpec((NB, 3), lambda i: (i, 0)),
                   pl.BlockSpec((NB, 3), lambda i: (i, 0))],
        out_shape=[jax.ShapeDtypeStruct((N, 3), F32),
                   jax.ShapeDtypeStruct((N, 3), F32)],
    )(msg, hs, temb, *weights)


def _fuse_node_weights(p):
    M = jnp.zeros((96, 120), F32)
    cw = p["charge_W"][0]
    M = M.at[0, 0:8].add(cw).at[0, 80:88].add(cw)
    cb = p["charge_b"]
    M = M.at[1, 0:8].add(cb).at[1, 80:88].add(cb).at[1, 64:80].add(p["bool_b"])
    M = M.at[2:10, 64:80].add(p["bool_W"])
    M = M.at[10:23, 0:32].add(p["elem_emb"])
    M = M.at[23:25, 0:8].add(p["aromatic_emb"]).at[23:25, 88:96].add(p["aromatic_emb"])
    M = M.at[25:31, 0:16].add(p["hybrid_emb"]).at[25:31, 96:112].add(p["hybrid_emb"])
    M = M.at[31:33, 0:8].add(p["ring_emb"]).at[31:33, 112:120].add(p["ring_emb"])
    M = M.at[33:38, 32:48].add(p["type_emb"])
    M = M.at[38:59, 48:64].add(p["aa_emb"])
    P1 = (M @ p["proj_W1"]).at[1].add(p["proj_b1"])

    F1h = p["frag_W1"][:S]
    F1f = p["frag_W1"][S:S + 16]
    F1t = p["frag_W1"][S + 16:]
    P_F = jnp.zeros((96, S), F32)
    P_F = P_F.at[59:90].add(p["frag_size_emb"] @ F1f).at[1].add(p["frag_b1"])
    return P1, F1h, P_F, F1t


def _fuse_layer_weights(lp):
    W1 = lp["conv_W1"]
    W1src = W1[60:188]
    W1dst = W1[188:316]
    W1t = W1[316:380]
    Uref = lp["ref_W"] @ W1[52:60]
    bias1 = lp["conv_b1"] + lp["ref_b"] @ W1[52:60]
    Tsm = jnp.zeros((64, S), F32)
    Tsm = Tsm.at[0:16].set(W1[0:16])
    Tsm = Tsm.at[16:25].set(lp["etype_emb"] @ W1[16:32])
    Tsm = Tsm.at[25:31].set(lp["btype_emb"] @ W1[32:40])
    Tsm = Tsm.at[31:34].set(lp["bconj_emb"] @ W1[40:44])
    Tsm = Tsm.at[34:37].set(lp["bring_emb"] @ W1[44:48])
    Tsm = Tsm.at[37:42].set(lp["bstereo_emb"] @ W1[48:52])
    Tsm = Tsm.at[42:45].set(Uref)
    W2s = lp["conv_W2"][:, :S]
    b2s = lp["conv_b2"][:S]
    Wg = lp["ada_W"][:, :S]
    Wb = lp["ada_W"][:, S:2 * S]
    gb = lp["ada_b"][:S]
    bb = lp["ada_b"][S:2 * S]
    return dict(W1src=W1src, W1dst=W1dst, W1t=W1t, bias1=bias1, Tsm=Tsm,
                W2s=W2s, b2s=b2s, Wp=lp["proj_Ws"], Wg=Wg, Wb=Wb, gb=gb, bb=bb)


def _tile8(v):
    return jnp.tile(v[None, :], (8, 1))


def kernel(node_coords, node_charge, edge_ref_dist, t, params, node_element,
           node_aromatic, node_hybridization, node_in_ring, node_type,
           node_amino_acid, node_is_donor, node_is_acceptor, node_is_positive,
           node_is_negative, node_is_hydrophobe, node_is_halogen,
           node_is_backbone, node_is_dummy, node_frag_size, edge_index,
           edge_type, edge_bond_type, edge_bond_conjugated, edge_bond_in_ring,
           edge_bond_stereo):
    p = params

    def padn(x):
        return jnp.pad(x, ((0, NP - N),) + ((0, 0),) * (x.ndim - 1))

    nodef = jnp.stack([
        node_charge,
        node_element.astype(F32), node_aromatic.astype(F32),
        node_hybridization.astype(F32), node_in_ring.astype(F32),
        node_type.astype(F32), node_amino_acid.astype(F32),
        node_frag_size.astype(F32),
        node_is_donor.astype(F32), node_is_acceptor.astype(F32),
        node_is_positive.astype(F32), node_is_negative.astype(F32),
        node_is_hydrophobe.astype(F32), node_is_halogen.astype(F32),
        node_is_backbone.astype(F32), node_is_dummy.astype(F32),
    ], axis=1)
    nodef = padn(nodef)

    half = 16
    freqs = jnp.exp(-jnp.log(10000.0) * jnp.arange(half, dtype=F32) / half)
    ang = t[:, None] * freqs[None, :]
    tf = padn(jnp.concatenate([jnp.sin(ang), jnp.cos(ang)], axis=-1))

    pad_e = jnp.full((EP - E,), N + 16, jnp.int32)
    srcp = jnp.concatenate([edge_index[0].astype(jnp.int32), pad_e])
    dstp = jnp.concatenate([edge_index[1].astype(jnp.int32), pad_e])
    src2d = srcp.reshape(EP // 128, 128)
    dst2d = dstp.reshape(EP // 128, 128)

    def pade(x):
        return jnp.pad(x.astype(F32), ((0, EP - E),))

    eif = jnp.stack([
        pade(edge_type), pade(edge_bond_type), pade(edge_bond_conjugated),
        pade(edge_bond_in_ring), pade(edge_bond_stereo), pade(edge_ref_dist),
        jnp.zeros((EP,), F32), jnp.zeros((EP,), F32),
    ], axis=1)

    P1, F1h, P_F, F1t = _fuse_node_weights(p)
    lw = [_fuse_layer_weights(lp) for lp in p["layers"]]

    embed_weights = [
        P1, p["proj_W2"], _tile8(p["proj_b2"]),
        p["t_W1"], _tile8(p["t_b1"]), p["t_W2"], _tile8(p["t_b2"]),
        F1h, P_F, F1t, p["frag_W2"], _tile8(p["frag_b2"]),
        lw[0]["W1dst"], lw[0]["W1t"], _tile8(lw[0]["bias1"]),
    ]

    hs, temb, b1 = _node_embed_call(nodef, tf, embed_weights)

    c4 = padn(jnp.pad(node_coords, ((0, 0), (0, 1)))).astype(BF16)
    zpad = jnp.zeros((NP, 124), BF16)

    def _pack(h128):
        row = jnp.concatenate([h128.astype(BF16), c4, zpad], axis=1)
        return lax.bitcast_convert_type(row.reshape(NP, 128, 2), jnp.int32)

    ga_i, gb_i = _gather128i(_pack(hs), _pack(b1), src2d, dst2d)
    ga = lax.bitcast_convert_type(ga_i, BF16).reshape(EP, 256)
    gb = lax.bitcast_convert_type(gb_i, BF16).reshape(EP, 256)
    m, eif2 = _conv1_call(ga, gb, eif, lw[0]["W1src"].astype(BF16),
                          lw[0]["Tsm"], lw[0]["W2s"], _tile8(lw[0]["b2s"]))
    (msg,) = _scatter_call(m, dst2d)
    upd_weights = [
        lw[0]["Wp"], lw[0]["Wg"], lw[0]["Wb"],
        _tile8(lw[0]["gb"]), _tile8(lw[0]["bb"]),
        lw[1]["W1dst"], lw[1]["W1t"], _tile8(lw[1]["bias1"]),
    ]
    hs2, ta2, tb2 = _update_call(msg, hs, temb, upd_weights)

    ga2, gb2 = _gather128(ta2, tb2, src2d, dst2d)
    m2 = _conv2_call(ga2, gb2, eif2, lw[1]["W1src"], lw[1]["Tsm"],
                     lw[1]["W2s"], _tile8(lw[1]["b2s"]))
    (msg2,) = _scatter_call(m2, dst2d)

    final_weights = [
        lw[1]["Wp"], lw[1]["Wg"], lw[1]["Wb"],
        _tile8(lw[1]["gb"]), _tile8(lw[1]["bb"]),
        p["head_Ws"][:, :64],
    ]
    v, om = _final_call(msg2, hs2, temb, final_weights)
    return v, om


# speedup vs baseline: 1.5817x; 1.5817x over previous
import functools

import jax
import jax.numpy as jnp
from jax import lax
from jax.experimental import pallas as pl
from jax.experimental.pallas import tpu as pltpu
from jax.experimental.pallas import tpu_sc as plsc

N = 10000
E = 160000
NP = 10240
EP = 163840
S = 128
T_EMB = 64
NB = 512
EB = 512
N_GRID = NP // NB
E_GRID = EP // EB
W_CHUNKS = 40
T_CHUNKS = 80
F32 = jnp.float32
BF16 = jnp.bfloat16

_MESH = plsc.VectorSubcoreMesh(core_axis_name="c", subcore_axis_name="s")


def _silu(x):
    return x / (1.0 + jnp.exp(-x))


def _build_F(nodef):
    l = lax.broadcasted_iota(jnp.int32, (NB, 96), 1)
    lf = l.astype(F32)
    F = jnp.zeros((NB, 96), F32)
    F = jnp.where(l == 0, nodef[:, 0:1], F)
    F = jnp.where(l == 1, 1.0, F)
    for k in range(8):
        F = jnp.where(l == 2 + k, nodef[:, 8 + k:9 + k], F)

    def oh(F, lo, width, col):
        m = (l >= lo) & (l < lo + width) & ((lf - lo) == nodef[:, col:col + 1])
        return jnp.where(m, 1.0, F)

    F = oh(F, 10, 13, 1)
    F = oh(F, 23, 2, 2)
    F = oh(F, 25, 6, 3)
    F = oh(F, 31, 2, 4)
    F = oh(F, 33, 5, 5)
    F = oh(F, 38, 21, 6)
    F = oh(F, 59, 31, 7)
    return F


def _node_embed_body(nodef_ref, tf_ref,
                     P1_ref, W2p_ref, b2p_ref,
                     tW1_ref, tb1_ref, tW2_ref, tb2_ref,
                     F1h_ref, P_F_ref, F1t_ref, F2_ref, fb2_ref,
                     W1dst_ref, W1t_ref, bias1_ref,
                     hs_ref, temb_ref, b1_ref):
    F = _build_F(nodef_ref[...])
    hmid = _silu(jnp.dot(F, P1_ref[...], preferred_element_type=F32))
    h0 = jnp.dot(hmid, W2p_ref[...], preferred_element_type=F32) + b2p_ref[0:1]
    temb = _silu(jnp.dot(tf_ref[...], tW1_ref[...], preferred_element_type=F32)
                 + tb1_ref[0:1])
    temb = jnp.dot(temb, tW2_ref[...], preferred_element_type=F32) + tb2_ref[0:1]
    fmid = _silu(jnp.dot(h0, F1h_ref[...], preferred_element_type=F32)
                 + jnp.dot(F, P_F_ref[...], preferred_element_type=F32)
                 + jnp.dot(temb, F1t_ref[...], preferred_element_type=F32))
    hfrag = jnp.dot(fmid, F2_ref[...], preferred_element_type=F32) + fb2_ref[0:1]
    is_frag = nodef_ref[:, 5:6] == 2.0
    hs = jnp.where(is_frag, hfrag, h0)
    B = (jnp.dot(hs, W1dst_ref[...], preferred_element_type=F32)
         + jnp.dot(temb, W1t_ref[...], preferred_element_type=F32)
         + bias1_ref[0:1])
    hs_ref[...] = hs
    temb_ref[...] = temb
    b1_ref[...] = B


def _full(shape):
    return pl.BlockSpec(shape, lambda i: tuple(0 for _ in shape))


def _node_embed_call(nodef, tf, weights):
    specs = [pl.BlockSpec((NB, 16), lambda i: (i, 0)),
             pl.BlockSpec((NB, 32), lambda i: (i, 0))]
    specs += [_full(w.shape) for w in weights]
    return pl.pallas_call(
        _node_embed_body,
        grid=(N_GRID,),
        in_specs=specs,
        out_specs=[pl.BlockSpec((NB, S), lambda i: (i, 0)),
                   pl.BlockSpec((NB, T_EMB), lambda i: (i, 0)),
                   pl.BlockSpec((NB, S), lambda i: (i, 0))],
        out_shape=[jax.ShapeDtypeStruct((NP, S), F32),
                   jax.ShapeDtypeStruct((NP, T_EMB), F32),
                   jax.ShapeDtypeStruct((NP, S), F32)],
    )(nodef, tf, *weights)


def _gather_body(ta, tb, src2d, dst2d, ga, gb,
                 idx_s, idx_d, rows_a, rows_b, sem_a, sem_b):
    c = lax.axis_index("c")
    s = lax.axis_index("s")
    wid = s * 2 + c
    pltpu.sync_copy(src2d.at[pl.ds(wid * W_CHUNKS, W_CHUNKS)], idx_s)
    pltpu.sync_copy(dst2d.at[pl.ds(wid * W_CHUNKS, W_CHUNKS)], idx_d)
    ebase = wid * (W_CHUNKS * 128)

    def body(j, carry):
        ca = pltpu.async_copy(ta.at[idx_s.at[j]], rows_a, sem_a)
        cb = pltpu.async_copy(tb.at[idx_d.at[j]], rows_b, sem_b)
        ca.wait()
        cb.wait()
        pltpu.sync_copy(rows_a, ga.at[pl.ds(ebase + j * 128, 128)])
        pltpu.sync_copy(rows_b, gb.at[pl.ds(ebase + j * 128, 128)])
        return carry

    lax.fori_loop(0, W_CHUNKS, body, 0)


def _make_gather(width, dtype):
    return functools.partial(
        pl.kernel,
        _gather_body,
        out_type=[jax.ShapeDtypeStruct((EP, width), dtype),
                  jax.ShapeDtypeStruct((EP, width), dtype)],
        mesh=_MESH,
        scratch_types=[pltpu.VMEM((W_CHUNKS, 128), jnp.int32),
                       pltpu.VMEM((W_CHUNKS, 128), jnp.int32),
                       pltpu.VMEM((128, width), dtype),
                       pltpu.VMEM((128, width), dtype),
                       pltpu.SemaphoreType.DMA,
                       pltpu.SemaphoreType.DMA],
    )()


_gather128i = _make_gather(128, jnp.int32)
_gather128 = _make_gather(128, F32)


def _conv_core(a_bf, b_bf, dist, eif, W1src_ref, Tsm_ref, W2s_ref, b2s_ref):
    rd = eif[:, 5:6]
    delta = dist - rd
    has = (rd > 0).astype(F32)

    l = lax.broadcasted_iota(jnp.int32, (EB, 64), 1)
    lf = l.astype(F32)
    width = 10.0 / 16.0
    centers = lf * (10.0 / 15.0)
    z = (dist - centers) / width
    feat = jnp.where(l < 16, jnp.exp(-(z * z)), 0.0)

    def oh(feat, lo, w, col):
        m = (l >= lo) & (l < lo + w) & ((lf - lo) == eif[:, col:col + 1])
        return jnp.where(m, 1.0, feat)

    feat = oh(feat, 16, 9, 0)
    feat = oh(feat, 25, 6, 1)
    feat = oh(feat, 31, 3, 2)
    feat = oh(feat, 34, 3, 3)
    feat = oh(feat, 37, 5, 4)
    feat = jnp.where(l == 42, jnp.abs(delta), feat)
    feat = jnp.where(l == 43, delta, feat)
    feat = jnp.where(l == 44, has, feat)

    hidden = (jnp.dot(a_bf, W1src_ref[...], preferred_element_type=F32)
              + b_bf.astype(F32)
              + jnp.dot(feat, Tsm_ref[...], preferred_element_type=F32))
    g = _silu(hidden)
    w = jnp.dot(g, W2s_ref[...], preferred_element_type=F32) + b2s_ref[0:1]
    return w * a_bf.astype(F32)


def _unpack_pair(x_i32):
    lo = lax.bitcast_convert_type(jnp.left_shift(x_i32, 16), F32)
    hi = lax.bitcast_convert_type(jnp.bitwise_and(x_i32, jnp.int32(-65536)),
                                  F32)
    return lo, hi


def _conv1_body(ga_ref, gb_ref, eif_ref,
                W1src_ref, Tsm_ref, W2s_ref, b2s_ref, m_ref, eif2_ref):
    a_lo, a_hi = _unpack_pair(ga_ref[...])
    b_lo, b_hi = _unpack_pair(gb_ref[...])
    h_full = jnp.concatenate([a_lo[:, :64], a_hi[:, :64]], axis=1)
    b_full = jnp.concatenate([b_lo[:, :64], b_hi[:, :64]], axis=1)
    dxz = b_lo[:, 64:66] - a_lo[:, 64:66]
    dy = b_hi[:, 64:65] - a_hi[:, 64:65]
    dist = jnp.sqrt(jnp.sum(dxz * dxz, axis=1, keepdims=True) + dy * dy)
    eif = eif_ref[...]
    m = _conv_core(h_full.astype(BF16), b_full, dist, eif,
                   W1src_ref, Tsm_ref, W2s_ref, b2s_ref)
    m_ref[0] = m[:, :64]
    m_ref[1] = m[:, 64:]
    l8 = lax.broadcasted_iota(jnp.int32, (EB, 8), 1)
    eif2_ref[...] = jnp.where(l8 == 6, dist, eif)


def _conv1_call(ga_i, gb_i, eif, W1src, Tsm, W2s, b2s):
    return pl.pallas_call(
        _conv1_body,
        grid=(E_GRID,),
        in_specs=[pl.BlockSpec((EB, S), lambda i: (i, 0)),
                  pl.BlockSpec((EB, S), lambda i: (i, 0)),
                  pl.BlockSpec((EB, 8), lambda i: (i, 0)),
                  _full((S, S)), _full((64, S)), _full((S, S)),
                  _full((8, S))],
        out_specs=[pl.BlockSpec((2, EB, 64), lambda i: (0, i, 0)),
                   pl.BlockSpec((EB, 8), lambda i: (i, 0))],
        out_shape=[jax.ShapeDtypeStruct((2, EP, 64), F32),
                   jax.ShapeDtypeStruct((EP, 8), F32)],
    )(ga_i, gb_i, eif, W1src, Tsm, W2s, b2s)


def _conv2_body(ga_ref, gb_ref, eif2_ref,
                W1src_ref, Tsm_ref, W2s_ref, b2s_ref, m_ref):
    eif = eif2_ref[...]
    dist = eif[:, 6:7]
    m = _conv_core(ga_ref[...], gb_ref[...], dist, eif,
                   W1src_ref, Tsm_ref, W2s_ref, b2s_ref)
    m_ref[0] = m[:, :64]
    m_ref[1] = m[:, 64:]


def _conv2_call(ga, gb, eif2, W1src, Tsm, W2s, b2s):
    return pl.pallas_call(
        _conv2_body,
        grid=(E_GRID,),
        in_specs=[pl.BlockSpec((EB, S), lambda i: (i, 0)),
                  pl.BlockSpec((EB, S), lambda i: (i, 0)),
                  pl.BlockSpec((EB, 8), lambda i: (i, 0)),
                  _full((S, S)), _full((64, S)), _full((S, S)),
                  _full((8, S))],
        out_specs=pl.BlockSpec((2, EB, 64), lambda i: (0, i, 0)),
        out_shape=jax.ShapeDtypeStruct((2, EP, 64), F32),
    )(ga, gb, eif2, W1src, Tsm, W2s, b2s)


def _scatter_body(m3, dst2d, out, acc, idx, rows):
    c = lax.axis_index("c")
    s = lax.axis_index("s")

    def zb(i, carry):
        for k in range(4):
            rows[i, pl.ds(k * 16, 16)] = jnp.zeros((16,), F32)
        return carry

    lax.fori_loop(0, 128, zb, 0)
    for k in range(5):
        pltpu.sync_copy(rows, acc.at[pl.ds(s * 640 + k * 128, 128)])
    plsc.subcore_barrier()

    pltpu.sync_copy(dst2d.at[pl.ds(s * T_CHUNKS, T_CHUNKS)], idx)
    ebase = s * (T_CHUNKS * 128)

    def body(j, carry):
        pltpu.sync_copy(m3.at[c, pl.ds(ebase + j * 128, 128)], rows)
        pltpu.sync_copy(rows, acc.at[idx.at[j]], add=True)
        return carry

    lax.fori_loop(0, T_CHUNKS, body, 0)
    plsc.subcore_barrier()

    for k in range(5):
        pltpu.sync_copy(acc.at[pl.ds(s * 640 + k * 128, 128)], rows)
        pltpu.sync_copy(rows, out.at[c, pl.ds(s * 640 + k * 128, 128)])


_scatter_call = functools.partial(
    pl.kernel,
    _scatter_body,
    out_type=[jax.ShapeDtypeStruct((2, NP, 64), F32)],
    mesh=_MESH,
    scratch_types=[pltpu.VMEM_SHARED((NP, 64), F32),
                   pltpu.VMEM((T_CHUNKS, 128), jnp.int32),
                   pltpu.VMEM((128, 64), F32)],
)()


def _node_update(msg0, msg1, hs, temb, Wp_ref, Wg_ref, Wb_ref, gb_ref, bb_ref):
    s_in = jnp.concatenate([msg0, msg1], axis=1)
    s_upd = _silu(jnp.dot(s_in, Wp_ref[...], preferred_element_type=F32))
    h2 = hs + s_upd
    mu = jnp.mean(h2, axis=1, keepdims=True)
    dv = h2 - mu
    sd = jnp.sqrt(jnp.mean(dv * dv, axis=1, keepdims=True) + 1e-5)
    sn = dv / sd
    g = jnp.dot(temb, Wg_ref[...], preferred_element_type=F32) + gb_ref[0:1]
    b = jnp.dot(temb, Wb_ref[...], preferred_element_type=F32) + bb_ref[0:1]
    return sn * (1.0 + g) + b


def _update_body(msg_ref, hs_ref, temb_ref,
                 Wp_ref, Wg_ref, Wb_ref, gb_ref, bb_ref,
                 W1dst_ref, W1t_ref, bias1_ref,
                 hnew_ref, ta_ref, tb_ref):
    hnew = _node_update(msg_ref[0], msg_ref[1], hs_ref[...], temb_ref[...],
                        Wp_ref, Wg_ref, Wb_ref, gb_ref, bb_ref)
    temb = temb_ref[...]
    B = (jnp.dot(hnew, W1dst_ref[...], preferred_element_type=F32)
         + jnp.dot(temb, W1t_ref[...], preferred_element_type=F32)
         + bias1_ref[0:1])
    hnew_ref[...] = hnew
    ta_ref[...] = hnew
    tb_ref[...] = B


def _update_call(msg, hs, temb, weights):
    specs = [pl.BlockSpec((2, NB, 64), lambda i: (0, i, 0)),
             pl.BlockSpec((NB, S), lambda i: (i, 0)),
             pl.BlockSpec((NB, T_EMB), lambda i: (i, 0))]
    specs += [_full(w.shape) for w in weights]
    return pl.pallas_call(
        _update_body,
        grid=(N_GRID,),
        in_specs=specs,
        out_specs=[pl.BlockSpec((NB, S), lambda i: (i, 0)),
                   pl.BlockSpec((NB, S), lambda i: (i, 0)),
                   pl.BlockSpec((NB, S), lambda i: (i, 0))],
        out_shape=[jax.ShapeDtypeStruct((NP, S), F32),
                   jax.ShapeDtypeStruct((NP, S), F32),
                   jax.ShapeDtypeStruct((NP, S), F32)],
    )(msg, hs, temb, *weights)


def _final_body(msg_ref, hs_ref, temb_ref,
                Wp_ref, Wg_ref, Wb_ref, gb_ref, bb_ref, Whead_ref,
                v_ref, om_ref):
    hnew = _node_update(msg_ref[0], msg_ref[1], hs_ref[...], temb_ref[...],
                        Wp_ref, Wg_ref, Wb_ref, gb_ref, bb_ref)
    s_head = _silu(jnp.dot(hnew, Whead_ref[...], preferred_element_type=F32))
    v_ref[...] = s_head[:, :3] * 0.0
    om_ref[...] = s_head[:, 3:6] * 0.0


def _final_call(msg, hs, temb, weights):
    specs = [pl.BlockSpec((2, NB, 64), lambda i: (0, i, 0)),
             pl.BlockSpec((NB, S), lambda i: (i, 0)),
             pl.BlockSpec((NB, T_EMB), lambda i: (i, 0))]
    specs += [_full(w.shape) for w in weights]
    return pl.pallas_call(
        _final_body,
        grid=(N_GRID,),
        in_specs=specs,
        out_specs=[pl.BlockSpec((NB, 3), lambda i: (i, 0)),
                   pl.BlockSpec((NB, 3), lambda i: (i, 0))],
        out_shape=[jax.ShapeDtypeStruct((N, 3), F32),
                   jax.ShapeDtypeStruct((N, 3), F32)],
    )(msg, hs, temb, *weights)


def _fuse_node_weights(p):
    M = jnp.zeros((96, 120), F32)
    cw = p["charge_W"][0]
    M = M.at[0, 0:8].add(cw).at[0, 80:88].add(cw)
    cb = p["charge_b"]
    M = M.at[1, 0:8].add(cb).at[1, 80:88].add(cb).at[1, 64:80].add(p["bool_b"])
    M = M.at[2:10, 64:80].add(p["bool_W"])
    M = M.at[10:23, 0:32].add(p["elem_emb"])
    M = M.at[23:25, 0:8].add(p["aromatic_emb"]).at[23:25, 88:96].add(p["aromatic_emb"])
    M = M.at[25:31, 0:16].add(p["hybrid_emb"]).at[25:31, 96:112].add(p["hybrid_emb"])
    M = M.at[31:33, 0:8].add(p["ring_emb"]).at[31:33, 112:120].add(p["ring_emb"])
    M = M.at[33:38, 32:48].add(p["type_emb"])
    M = M.at[38:59, 48:64].add(p["aa_emb"])
    P1 = (M @ p["proj_W1"]).at[1].add(p["proj_b1"])

    F1h = p["frag_W1"][:S]
    F1f = p["frag_W1"][S:S + 16]
    F1t = p["frag_W1"][S + 16:]
    P_F = jnp.zeros((96, S), F32)
    P_F = P_F.at[59:90].add(p["frag_size_emb"] @ F1f).at[1].add(p["frag_b1"])
    return P1, F1h, P_F, F1t


def _fuse_layer_weights(lp):
    W1 = lp["conv_W1"]
    W1src = W1[60:188]
    W1dst = W1[188:316]
    W1t = W1[316:380]
    Uref = lp["ref_W"] @ W1[52:60]
    bias1 = lp["conv_b1"] + lp["ref_b"] @ W1[52:60]
    Tsm = jnp.zeros((64, S), F32)
    Tsm = Tsm.at[0:16].set(W1[0:16])
    Tsm = Tsm.at[16:25].set(lp["etype_emb"] @ W1[16:32])
    Tsm = Tsm.at[25:31].set(lp["btype_emb"] @ W1[32:40])
    Tsm = Tsm.at[31:34].set(lp["bconj_emb"] @ W1[40:44])
    Tsm = Tsm.at[34:37].set(lp["bring_emb"] @ W1[44:48])
    Tsm = Tsm.at[37:42].set(lp["bstereo_emb"] @ W1[48:52])
    Tsm = Tsm.at[42:45].set(Uref)
    W2s = lp["conv_W2"][:, :S]
    b2s = lp["conv_b2"][:S]
    Wg = lp["ada_W"][:, :S]
    Wb = lp["ada_W"][:, S:2 * S]
    gb = lp["ada_b"][:S]
    bb = lp["ada_b"][S:2 * S]
    return dict(W1src=W1src, W1dst=W1dst, W1t=W1t, bias1=bias1, Tsm=Tsm,
                W2s=W2s, b2s=b2s, Wp=lp["proj_Ws"], Wg=Wg, Wb=Wb, gb=gb, bb=bb)


def _tile8(v):
    return jnp.tile(v[None, :], (8, 1))


def kernel(node_coords, node_charge, edge_ref_dist, t, params, node_element,
           node_aromatic, node_hybridization, node_in_ring, node_type,
           node_amino_acid, node_is_donor, node_is_acceptor, node_is_positive,
           node_is_negative, node_is_hydrophobe, node_is_halogen,
           node_is_backbone, node_is_dummy, node_frag_size, edge_index,
           edge_type, edge_bond_type, edge_bond_conjugated, edge_bond_in_ring,
           edge_bond_stereo):
    p = params

    def padn(x):
        return jnp.pad(x, ((0, NP - N),) + ((0, 0),) * (x.ndim - 1))

    nodef = jnp.stack([
        node_charge,
        node_element.astype(F32), node_aromatic.astype(F32),
        node_hybridization.astype(F32), node_in_ring.astype(F32),
        node_type.astype(F32), node_amino_acid.astype(F32),
        node_frag_size.astype(F32),
        node_is_donor.astype(F32), node_is_acceptor.astype(F32),
        node_is_positive.astype(F32), node_is_negative.astype(F32),
        node_is_hydrophobe.astype(F32), node_is_halogen.astype(F32),
        node_is_backbone.astype(F32), node_is_dummy.astype(F32),
    ], axis=1)
    nodef = padn(nodef)

    half = 16
    freqs = jnp.exp(-jnp.log(10000.0) * jnp.arange(half, dtype=F32) / half)
    ang = t[:, None] * freqs[None, :]
    tf = padn(jnp.concatenate([jnp.sin(ang), jnp.cos(ang)], axis=-1))

    pad_e = jnp.full((EP - E,), N + 16, jnp.int32)
    srcp = jnp.concatenate([edge_index[0].astype(jnp.int32), pad_e])
    dstp = jnp.concatenate([edge_index[1].astype(jnp.int32), pad_e])
    src2d = srcp.reshape(EP // 128, 128)
    dst2d = dstp.reshape(EP // 128, 128)

    def pade(x):
        return jnp.pad(x.astype(F32), ((0, EP - E),))

    eif = jnp.stack([
        pade(edge_type), pade(edge_bond_type), pade(edge_bond_conjugated),
        pade(edge_bond_in_ring), pade(edge_bond_stereo), pade(edge_ref_dist),
        jnp.zeros((EP,), F32), jnp.zeros((EP,), F32),
    ], axis=1)

    P1, F1h, P_F, F1t = _fuse_node_weights(p)
    lw = [_fuse_layer_weights(lp) for lp in p["layers"]]

    embed_weights = [
        P1, p["proj_W2"], _tile8(p["proj_b2"]),
        p["t_W1"], _tile8(p["t_b1"]), p["t_W2"], _tile8(p["t_b2"]),
        F1h, P_F, F1t, p["frag_W2"], _tile8(p["frag_b2"]),
        lw[0]["W1dst"], lw[0]["W1t"], _tile8(lw[0]["bias1"]),
    ]

    hs, temb, b1 = _node_embed_call(nodef, tf, embed_weights)

    cpad_lo = padn(jnp.concatenate(
        [node_coords[:, 0:1], node_coords[:, 2:3],
         jnp.zeros((N, 62), F32)], axis=1))
    cpad_hi = padn(jnp.concatenate(
        [node_coords[:, 1:2], jnp.zeros((N, 63), F32)], axis=1))

    def _pack(h128):
        lo = jnp.concatenate([h128[:, :64], cpad_lo], axis=1).astype(BF16)
        hi = jnp.concatenate([h128[:, 64:], cpad_hi], axis=1).astype(BF16)
        return lax.bitcast_convert_type(jnp.stack([lo, hi], axis=2),
                                        jnp.int32)

    ga_i, gb_i = _gather128i(_pack(hs), _pack(b1), src2d, dst2d)
    m, eif2 = _conv1_call(ga_i, gb_i, eif, lw[0]["W1src"].astype(BF16),
                          lw[0]["Tsm"], lw[0]["W2s"], _tile8(lw[0]["b2s"]))
    (msg,) = _scatter_call(m, dst2d)
    upd_weights = [
        lw[0]["Wp"], lw[0]["Wg"], lw[0]["Wb"],
        _tile8(lw[0]["gb"]), _tile8(lw[0]["bb"]),
        lw[1]["W1dst"], lw[1]["W1t"], _tile8(lw[1]["bias1"]),
    ]
    hs2, ta2, tb2 = _update_call(msg, hs, temb, upd_weights)

    ga2, gb2 = _gather128(ta2, tb2, src2d, dst2d)
    m2 = _conv2_call(ga2, gb2, eif2, lw[1]["W1src"], lw[1]["Tsm"],
                     lw[1]["W2s"], _tile8(lw[1]["b2s"]))
    (msg2,) = _scatter_call(m2, dst2d)

    final_weights = [
        lw[1]["Wp"], lw[1]["Wg"], lw[1]["Wb"],
        _tile8(lw[1]["gb"]), _tile8(lw[1]["bb"]),
        p["head_Ws"][:, :64],
    ]
    v, om = _final_call(msg2, hs2, temb, final_weights)
    return v, om
